# baseline (speedup 1.0000x reference)
import functools

import numpy as np

import jax
import jax.numpy as jnp
from jax.experimental import pallas as pl
from jax.experimental.pallas import tpu as pltpu


def _stats_kernel(x_ref, w_ref, b_ref, g_ref, be_ref, sc_ref, sh_ref,
                  acc_ref, *, nimg, G, Co, cnt, eps):
    g = pl.program_id(0)

    @pl.when(g == 0)
    def _():
        acc_ref[...] = jnp.zeros_like(acc_ref)

    w = w_ref[...]
    b = b_ref[...]
    acc = None
    for i in range(nimg):
        x = x_ref[i].astype(jnp.bfloat16)
        z = jnp.dot(w, x, preferred_element_type=jnp.float32)
        z = jnp.maximum(z + b, 0.0)
        st = jnp.concatenate(
            [jnp.sum(z, axis=1, keepdims=True),
             jnp.sum(z * z, axis=1, keepdims=True)], axis=1)
        acc = st if acc is None else acc + st
    acc_ref[...] += acc

    @pl.when(g == G - 1)
    def _():
        st = acc_ref[...]
        stc = (st[0 * Co:1 * Co] + st[1 * Co:2 * Co]
               + st[2 * Co:3 * Co] + st[3 * Co:4 * Co])
        mean = stc[:, 0:1] / cnt
        var = jnp.maximum(stc[:, 1:2] / cnt - mean * mean, 0.0)
        scale = g_ref[...] * jax.lax.rsqrt(var + eps)
        shift = be_ref[...] - mean * scale
        sc_ref[...] = jnp.concatenate([scale] * 4, axis=0)
        sh_ref[...] = jnp.concatenate([shift] * 4, axis=0)


def _convt_stats(x3, w_taps, b_taps, g1c, be1c, nimg, cnt, eps):
    N, Ci, HWc = x3.shape
    Ko = w_taps.shape[0]
    Co = Ko // 4
    G = N // nimg
    body = functools.partial(_stats_kernel, nimg=nimg, G=G, Co=Co,
                             cnt=cnt, eps=eps)
    return pl.pallas_call(
        body,
        grid=(G,),
        in_specs=[
            pl.BlockSpec((nimg, Ci, HWc), lambda g: (g, 0, 0)),
            pl.BlockSpec((Ko, Ci), lambda g: (0, 0)),
            pl.BlockSpec((Ko, 1), lambda g: (0, 0)),
            pl.BlockSpec((Co, 1), lambda g: (0, 0)),
            pl.BlockSpec((Co, 1), lambda g: (0, 0)),
        ],
        out_specs=(
            pl.BlockSpec((Ko, 1), lambda g: (0, 0)),
            pl.BlockSpec((Ko, 1), lambda g: (0, 0)),
        ),
        out_shape=(
            jax.ShapeDtypeStruct((Ko, 1), jnp.float32),
            jax.ShapeDtypeStruct((Ko, 1), jnp.float32),
        ),
        scratch_shapes=[pltpu.VMEM((Ko, 2), jnp.float32)],
        compiler_params=pltpu.CompilerParams(
            dimension_semantics=("arbitrary",)),
    )(x3, w_taps, b_taps, g1c, be1c)


def _main_kernel(x_ref, skip_ref, sc1_ref, sh1_ref,
                 wt_ref, bt_ref, perm_ref,
                 w1u_ref, w1s_ref, b1_ref, w2_ref, b2_ref,
                 g2_ref, be2_ref,
                 y_ref, sc2_ref, sh2_ref, acc_ref,
                 *, Co, H, W, nimg, G, cnt, eps):
    HW = H * W
    step = pl.program_id(0)
    offs = [(dy, dx) for dy in (-1, 0, 1) for dx in (-1, 0, 1)]

    @pl.when(step == 0)
    def _():
        acc_ref[...] = jnp.zeros_like(acc_ref)

    col = jax.lax.broadcasted_iota(jnp.int32, (1, HW), 1)
    xi = col % W
    yi = col // W
    masks = {}
    for dy, dx in offs:
        if (dy, dx) == (0, 0):
            continue
        m = ((xi + dx >= 0) & (xi + dx < W) &
             (yi + dy >= 0) & (yi + dy < H))
        masks[(dy, dx)] = m.astype(jnp.bfloat16)

    def patches(img):
        taps = []
        for dy, dx in offs:
            if (dy, dx) == (0, 0):
                taps.append(img)
                continue
            s = dy * W + dx
            shifted = pltpu.roll(img, shift=(-s) % HW, axis=1)
            taps.append(shifted * masks[(dy, dx)])
        return jnp.concatenate(taps, axis=0)

    sc1 = sc1_ref[...]
    sh1 = sh1_ref[...]
    acc = None
    for i in range(nimg):
        x = x_ref[i].astype(jnp.bfloat16)
        z = jnp.dot(wt_ref[...], x, preferred_element_type=jnp.float32)
        z = jnp.maximum(z + bt_ref[...], 0.0)
        zn = (z * sc1 + sh1).astype(jnp.bfloat16)
        u = jnp.dot(zn[0 * Co:1 * Co], perm_ref[0],
                    preferred_element_type=jnp.float32)
        for p in range(1, 4):
            u = u + jnp.dot(zn[p * Co:(p + 1) * Co], perm_ref[p],
                            preferred_element_type=jnp.float32)
        u = u.astype(jnp.bfloat16)

        mid = (jnp.dot(w1u_ref[...], patches(u),
                       preferred_element_type=jnp.float32)
               + jnp.dot(w1s_ref[...], patches(skip_ref[i].astype(jnp.bfloat16)),
                         preferred_element_type=jnp.float32))
        mid = jnp.maximum(mid + b1_ref[...], 0.0).astype(jnp.bfloat16)
        y = jnp.dot(w2_ref[...], patches(mid),
                    preferred_element_type=jnp.float32)
        y = jnp.maximum(y + b2_ref[...], 0.0)

        y_ref[i] = y.astype(jnp.bfloat16)
        st = jnp.concatenate(
            [jnp.sum(y, axis=1, keepdims=True),
             jnp.sum(y * y, axis=1, keepdims=True)], axis=1)
        acc = st if acc is None else acc + st
    acc_ref[...] += acc

    @pl.when(step == G - 1)
    def _():
        st = acc_ref[...]
        mean = st[:, 0:1] / cnt
        var = jnp.maximum(st[:, 1:2] / cnt - mean * mean, 0.0)
        scale = g2_ref[...] * jax.lax.rsqrt(var + eps)
        sc2_ref[...] = scale
        sh2_ref[...] = be2_ref[...] - mean * scale


def _fused_block(x3, skip3, sc1, sh1, w_taps, b_taps, perm,
                 w1mu, w1ms, b1c, w2m, b2c, g2c, be2c,
                 H, W, nimg, cnt, eps):
    N, Ci, HWc = x3.shape
    Cs = skip3.shape[1]
    Ko = w_taps.shape[0]
    Co = Ko // 4
    Cm = w1mu.shape[0]
    HW = H * W
    G = N // nimg

    est = (2 * nimg * (Ci * HWc + Cs * HW + Co * HW) * 4
           + (Ko * Ci + Cm * 9 * (Co + Cs) + Co * 9 * Cm + 4 * HWc * HW) * 2
           + (9 * (Co + Cs) + 9 * Cm) * HW * 2
           + (Ko * HWc + Co * HW + Cm * HW) * 4)
    vmem_limit = int(min(max(3 * est, 32 * 2 ** 20), 100 * 2 ** 20))

    body = functools.partial(_main_kernel, Co=Co, H=H, W=W, nimg=nimg, G=G,
                             cnt=cnt, eps=eps)
    return pl.pallas_call(
        body,
        grid=(G,),
        in_specs=[
            pl.BlockSpec((nimg, Ci, HWc), lambda g: (g, 0, 0)),
            pl.BlockSpec((nimg, Cs, HW), lambda g: (g, 0, 0)),
            pl.BlockSpec((Ko, 1), lambda g: (0, 0)),
            pl.BlockSpec((Ko, 1), lambda g: (0, 0)),
            pl.BlockSpec((Ko, Ci), lambda g: (0, 0)),
            pl.BlockSpec((Ko, 1), lambda g: (0, 0)),
            pl.BlockSpec((4, HWc, HW), lambda g: (0, 0, 0)),
            pl.BlockSpec((Cm, 9 * Co), lambda g: (0, 0)),
            pl.BlockSpec((Cm, 9 * Cs), lambda g: (0, 0)),
            pl.BlockSpec((Cm, 1), lambda g: (0, 0)),
            pl.BlockSpec((Co, 9 * Cm), lambda g: (0, 0)),
            pl.BlockSpec((Co, 1), lambda g: (0, 0)),
            pl.BlockSpec((Co, 1), lambda g: (0, 0)),
            pl.BlockSpec((Co, 1), lambda g: (0, 0)),
        ],
        out_specs=(
            pl.BlockSpec((nimg, Co, HW), lambda g: (g, 0, 0)),
            pl.BlockSpec((Co, 1), lambda g: (0, 0)),
            pl.BlockSpec((Co, 1), lambda g: (0, 0)),
        ),
        out_shape=(
            jax.ShapeDtypeStruct((N, Co, HW), jnp.bfloat16),
            jax.ShapeDtypeStruct((Co, 1), jnp.float32),
            jax.ShapeDtypeStruct((Co, 1), jnp.float32),
        ),
        scratch_shapes=[pltpu.VMEM((Co, 2), jnp.float32)],
        compiler_params=pltpu.CompilerParams(
            dimension_semantics=("arbitrary",),
            vmem_limit_bytes=vmem_limit),
    )(x3, skip3, sc1, sh1, w_taps, b_taps, perm,
      w1mu, w1ms, b1c, w2m, b2c, g2c, be2c)


def _affine_kernel(y_ref, sc_ref, sh_ref, o_ref, *, nimg, H, W):
    Co = y_ref.shape[1]
    sc = sc_ref[...]
    sh = sh_ref[...]
    for i in range(nimg):
        o = y_ref[i].astype(jnp.float32) * sc + sh
        o_ref[i] = o.reshape(Co, H, W)


def _final_affine(y, sc2, sh2, nimg, H, W):
    N, Co, HW = y.shape
    G = N // nimg
    body = functools.partial(_affine_kernel, nimg=nimg, H=H, W=W)
    return pl.pallas_call(
        body,
        grid=(G,),
        in_specs=[
            pl.BlockSpec((nimg, Co, HW), lambda g: (g, 0, 0)),
            pl.BlockSpec((Co, 1), lambda g: (0, 0)),
            pl.BlockSpec((Co, 1), lambda g: (0, 0)),
        ],
        out_specs=pl.BlockSpec((nimg, Co, H, W), lambda g: (g, 0, 0, 0)),
        out_shape=jax.ShapeDtypeStruct((N, Co, H, W), jnp.float32),
        compiler_params=pltpu.CompilerParams(
            dimension_semantics=("arbitrary",)),
    )(y, sc2, sh2)


@functools.lru_cache(maxsize=4)
def _perm_mats(Hc, Wc):
    H2, W2 = 2 * Hc, 2 * Wc
    HWc = Hc * Wc
    P = np.zeros((4, HWc, H2 * W2), np.float32)
    hw = np.arange(HWc)
    h, w = hw // Wc, hw % Wc
    for dy in (0, 1):
        for dx in (0, 1):
            P[dy * 2 + dx, hw, (2 * h + dy) * W2 + 2 * w + dx] = 1.0
    return P


def kernel(x, skip, wt, bt, w1, b1, w2, b2, g1, be1, g2, be2):
    eps = 1e-5
    mxu_dtype = jnp.bfloat16
    x = x.astype(jnp.float32)
    skip = skip.astype(jnp.float32)
    N, Ci, Hc, Wc = x.shape
    _, Cs, H2, W2 = skip.shape
    Co = wt.shape[1]
    Cm = w1.shape[0]
    HWc, HW2 = Hc * Wc, H2 * W2

    x3 = x.reshape(N, Ci, HWc)
    skip3 = skip.reshape(N, Cs, HW2)

    w_taps = (jnp.transpose(wt, (2, 3, 1, 0))
              .reshape(4 * Co, Ci).astype(mxu_dtype))
    b_taps = jnp.tile(bt, 4).reshape(4 * Co, 1)

    cnt = N * HW2
    n1 = 8 if N % 8 == 0 else 1
    sc1, sh1 = _convt_stats(x3, w_taps, b_taps,
                            g1.reshape(Co, 1), be1.reshape(Co, 1),
                            n1, cnt, eps)

    perm = jnp.asarray(_perm_mats(Hc, Wc), dtype=mxu_dtype)
    w1mu = (jnp.transpose(w1[:, :Co], (0, 2, 3, 1))
            .reshape(Cm, 9 * Co).astype(mxu_dtype))
    w1ms = (jnp.transpose(w1[:, Co:], (0, 2, 3, 1))
            .reshape(Cm, 9 * Cs).astype(mxu_dtype))
    w2m = jnp.transpose(w2, (0, 2, 3, 1)).reshape(Co, 9 * Cm).astype(mxu_dtype)
    b1c = b1.reshape(Cm, 1)
    b2c = b2.reshape(Co, 1)

    n2 = 2 if N % 2 == 0 else 1
    y, sc2, sh2 = _fused_block(x3, skip3, sc1, sh1, w_taps, b_taps, perm,
                               w1mu, w1ms, b1c, w2m, b2c,
                               g2.reshape(Co, 1), be2.reshape(Co, 1),
                               H2, W2, n2, cnt, eps)

    return _final_affine(y, sc2, sh2, n1, H2, W2)


# speedup vs baseline: 1.3298x; 1.3298x over previous
import functools

import numpy as np

import jax
import jax.numpy as jnp
from jax.experimental import pallas as pl
from jax.experimental.pallas import tpu as pltpu


def _stats_kernel(x_ref, w_ref, b_ref, g_ref, be_ref, sc_ref, sh_ref,
                  acc_ref, *, nimg, G, Co, cnt, eps):
    g = pl.program_id(0)

    @pl.when(g == 0)
    def _():
        acc_ref[...] = jnp.zeros_like(acc_ref)

    w = w_ref[...]
    b = b_ref[...]
    acc = None
    for i in range(nimg):
        x = x_ref[i].astype(jnp.bfloat16)
        z = jnp.dot(w, x, preferred_element_type=jnp.float32)
        z = jnp.maximum(z + b, 0.0)
        st = jnp.concatenate(
            [jnp.sum(z, axis=1, keepdims=True),
             jnp.sum(z * z, axis=1, keepdims=True)], axis=1)
        acc = st if acc is None else acc + st
    acc_ref[...] += acc

    @pl.when(g == G - 1)
    def _():
        st = acc_ref[...]
        stc = (st[0 * Co:1 * Co] + st[1 * Co:2 * Co]
               + st[2 * Co:3 * Co] + st[3 * Co:4 * Co])
        mean = stc[:, 0:1] / cnt
        var = jnp.maximum(stc[:, 1:2] / cnt - mean * mean, 0.0)
        scale = g_ref[...] * jax.lax.rsqrt(var + eps)
        shift = be_ref[...] - mean * scale
        sc_ref[...] = jnp.concatenate([scale] * 4, axis=0)
        sh_ref[...] = jnp.concatenate([shift] * 4, axis=0)


def _convt_stats(x3, w_taps, b_taps, g1c, be1c, nimg, cnt, eps):
    N, Ci, HWc = x3.shape
    Ko = w_taps.shape[0]
    Co = Ko // 4
    G = N // nimg
    body = functools.partial(_stats_kernel, nimg=nimg, G=G, Co=Co,
                             cnt=cnt, eps=eps)
    return pl.pallas_call(
        body,
        grid=(G,),
        in_specs=[
            pl.BlockSpec((nimg, Ci, HWc), lambda g: (g, 0, 0)),
            pl.BlockSpec((Ko, Ci), lambda g: (0, 0)),
            pl.BlockSpec((Ko, 1), lambda g: (0, 0)),
            pl.BlockSpec((Co, 1), lambda g: (0, 0)),
            pl.BlockSpec((Co, 1), lambda g: (0, 0)),
        ],
        out_specs=(
            pl.BlockSpec((Ko, 1), lambda g: (0, 0)),
            pl.BlockSpec((Ko, 1), lambda g: (0, 0)),
        ),
        out_shape=(
            jax.ShapeDtypeStruct((Ko, 1), jnp.float32),
            jax.ShapeDtypeStruct((Ko, 1), jnp.float32),
        ),
        scratch_shapes=[pltpu.VMEM((Ko, 2), jnp.float32)],
        compiler_params=pltpu.CompilerParams(
            dimension_semantics=("arbitrary",)),
    )(x3, w_taps, b_taps, g1c, be1c)


def _main_kernel(x_ref, skip_ref, sc1_ref, sh1_ref,
                 wt_ref, bt_ref, perm_ref,
                 w1u_ref, w1s_ref, b1_ref, w2_ref, b2_ref,
                 g2_ref, be2_ref,
                 y_ref, sc2_ref, sh2_ref, acc_ref,
                 *, Co, H, W, nimg, G, cnt, eps):
    HW = H * W
    step = pl.program_id(0)
    offs = [(dy, dx) for dy in (-1, 0, 1) for dx in (-1, 0, 1)]

    @pl.when(step == 0)
    def _():
        acc_ref[...] = jnp.zeros_like(acc_ref)

    col = jax.lax.broadcasted_iota(jnp.int32, (1, HW), 1)
    xi = col % W
    yi = col // W
    masks = {}
    for dy, dx in offs:
        if (dy, dx) == (0, 0):
            continue
        m = ((xi + dx >= 0) & (xi + dx < W) &
             (yi + dy >= 0) & (yi + dy < H))
        masks[(dy, dx)] = m.astype(jnp.bfloat16)

    def patches(img):
        taps = []
        for dy, dx in offs:
            if (dy, dx) == (0, 0):
                taps.append(img)
                continue
            s = dy * W + dx
            shifted = pltpu.roll(img, shift=(-s) % HW, axis=1)
            taps.append(shifted * masks[(dy, dx)])
        return jnp.concatenate(taps, axis=0)

    sc1 = sc1_ref[...]
    sh1 = sh1_ref[...]
    acc = None
    for i in range(nimg):
        x = x_ref[i].astype(jnp.bfloat16)
        z = jnp.dot(wt_ref[...], x, preferred_element_type=jnp.float32)
        z = jnp.maximum(z + bt_ref[...], 0.0)
        zn = (z * sc1 + sh1).astype(jnp.bfloat16)
        u = jnp.dot(zn[0 * Co:1 * Co], perm_ref[0],
                    preferred_element_type=jnp.float32)
        for p in range(1, 4):
            u = u + jnp.dot(zn[p * Co:(p + 1) * Co], perm_ref[p],
                            preferred_element_type=jnp.float32)
        u = u.astype(jnp.bfloat16)

        mid = (jnp.dot(w1u_ref[...], patches(u),
                       preferred_element_type=jnp.float32)
               + jnp.dot(w1s_ref[...], patches(skip_ref[i].astype(jnp.bfloat16)),
                         preferred_element_type=jnp.float32))
        mid = jnp.maximum(mid + b1_ref[...], 0.0).astype(jnp.bfloat16)
        y = jnp.dot(w2_ref[...], patches(mid),
                    preferred_element_type=jnp.float32)
        y = jnp.maximum(y + b2_ref[...], 0.0)

        y_ref[i] = y.astype(jnp.bfloat16)
        st = jnp.concatenate(
            [jnp.sum(y, axis=1, keepdims=True),
             jnp.sum(y * y, axis=1, keepdims=True)], axis=1)
        acc = st if acc is None else acc + st
    acc_ref[...] += acc

    @pl.when(step == G - 1)
    def _():
        st = acc_ref[...]
        mean = st[:, 0:1] / cnt
        var = jnp.maximum(st[:, 1:2] / cnt - mean * mean, 0.0)
        scale = g2_ref[...] * jax.lax.rsqrt(var + eps)
        sc2_ref[...] = scale
        sh2_ref[...] = be2_ref[...] - mean * scale


def _fused_block(x3, skip3, sc1, sh1, w_taps, b_taps, perm,
                 w1mu, w1ms, b1c, w2m, b2c, g2c, be2c,
                 H, W, nimg, cnt, eps):
    N, Ci, HWc = x3.shape
    Cs = skip3.shape[1]
    Ko = w_taps.shape[0]
    Co = Ko // 4
    Cm = w1mu.shape[0]
    HW = H * W
    G = N // nimg

    est = (2 * nimg * (Ci * HWc + Cs * HW + Co * HW) * 4
           + (Ko * Ci + Cm * 9 * (Co + Cs) + Co * 9 * Cm + 4 * HWc * HW) * 2
           + (9 * (Co + Cs) + 9 * Cm) * HW * 2
           + (Ko * HWc + Co * HW + Cm * HW) * 4)
    vmem_limit = int(min(max(3 * est, 32 * 2 ** 20), 100 * 2 ** 20))

    body = functools.partial(_main_kernel, Co=Co, H=H, W=W, nimg=nimg, G=G,
                             cnt=cnt, eps=eps)
    return pl.pallas_call(
        body,
        grid=(G,),
        in_specs=[
            pl.BlockSpec((nimg, Ci, HWc), lambda g: (g, 0, 0)),
            pl.BlockSpec((nimg, Cs, HW), lambda g: (g, 0, 0)),
            pl.BlockSpec((Ko, 1), lambda g: (0, 0)),
            pl.BlockSpec((Ko, 1), lambda g: (0, 0)),
            pl.BlockSpec((Ko, Ci), lambda g: (0, 0)),
            pl.BlockSpec((Ko, 1), lambda g: (0, 0)),
            pl.BlockSpec((4, HWc, HW), lambda g: (0, 0, 0)),
            pl.BlockSpec((Cm, 9 * Co), lambda g: (0, 0)),
            pl.BlockSpec((Cm, 9 * Cs), lambda g: (0, 0)),
            pl.BlockSpec((Cm, 1), lambda g: (0, 0)),
            pl.BlockSpec((Co, 9 * Cm), lambda g: (0, 0)),
            pl.BlockSpec((Co, 1), lambda g: (0, 0)),
            pl.BlockSpec((Co, 1), lambda g: (0, 0)),
            pl.BlockSpec((Co, 1), lambda g: (0, 0)),
        ],
        out_specs=(
            pl.BlockSpec((nimg, Co, HW), lambda g: (g, 0, 0)),
            pl.BlockSpec((Co, 1), lambda g: (0, 0)),
            pl.BlockSpec((Co, 1), lambda g: (0, 0)),
        ),
        out_shape=(
            jax.ShapeDtypeStruct((N, Co, HW), jnp.bfloat16),
            jax.ShapeDtypeStruct((Co, 1), jnp.float32),
            jax.ShapeDtypeStruct((Co, 1), jnp.float32),
        ),
        scratch_shapes=[pltpu.VMEM((Co, 2), jnp.float32)],
        compiler_params=pltpu.CompilerParams(
            dimension_semantics=("arbitrary",),
            vmem_limit_bytes=vmem_limit),
    )(x3, skip3, sc1, sh1, w_taps, b_taps, perm,
      w1mu, w1ms, b1c, w2m, b2c, g2c, be2c)


def _affine_kernel(y_ref, sc_ref, sh_ref, o_ref):
    o_ref[...] = (y_ref[...].astype(jnp.float32) * sc_ref[...][None]
                  + sh_ref[...][None])


def _final_affine(y, sc2, sh2, nimg):
    N, Co, HW = y.shape
    G = N // nimg
    return pl.pallas_call(
        _affine_kernel,
        grid=(G,),
        in_specs=[
            pl.BlockSpec((nimg, Co, HW), lambda g: (g, 0, 0)),
            pl.BlockSpec((Co, 1), lambda g: (0, 0)),
            pl.BlockSpec((Co, 1), lambda g: (0, 0)),
        ],
        out_specs=pl.BlockSpec((nimg, Co, HW), lambda g: (g, 0, 0)),
        out_shape=jax.ShapeDtypeStruct((N, Co, HW), jnp.float32),
        compiler_params=pltpu.CompilerParams(
            dimension_semantics=("arbitrary",)),
    )(y, sc2, sh2)


@functools.lru_cache(maxsize=4)
def _perm_mats(Hc, Wc):
    H2, W2 = 2 * Hc, 2 * Wc
    HWc = Hc * Wc
    P = np.zeros((4, HWc, H2 * W2), np.float32)
    hw = np.arange(HWc)
    h, w = hw // Wc, hw % Wc
    for dy in (0, 1):
        for dx in (0, 1):
            P[dy * 2 + dx, hw, (2 * h + dy) * W2 + 2 * w + dx] = 1.0
    return P


def kernel(x, skip, wt, bt, w1, b1, w2, b2, g1, be1, g2, be2):
    eps = 1e-5
    mxu_dtype = jnp.bfloat16
    x = x.astype(jnp.float32)
    skip = skip.astype(jnp.float32)
    N, Ci, Hc, Wc = x.shape
    _, Cs, H2, W2 = skip.shape
    Co = wt.shape[1]
    Cm = w1.shape[0]
    HWc, HW2 = Hc * Wc, H2 * W2

    x3 = x.reshape(N, Ci, HWc)
    skip3 = skip.reshape(N, Cs, HW2)

    w_taps = (jnp.transpose(wt, (2, 3, 1, 0))
              .reshape(4 * Co, Ci).astype(mxu_dtype))
    b_taps = jnp.tile(bt, 4).reshape(4 * Co, 1)

    cnt = N * HW2
    n1 = 8 if N % 8 == 0 else 1
    sc1, sh1 = _convt_stats(x3, w_taps, b_taps,
                            g1.reshape(Co, 1), be1.reshape(Co, 1),
                            n1, cnt, eps)

    perm = jnp.asarray(_perm_mats(Hc, Wc), dtype=mxu_dtype)
    w1mu = (jnp.transpose(w1[:, :Co], (0, 2, 3, 1))
            .reshape(Cm, 9 * Co).astype(mxu_dtype))
    w1ms = (jnp.transpose(w1[:, Co:], (0, 2, 3, 1))
            .reshape(Cm, 9 * Cs).astype(mxu_dtype))
    w2m = jnp.transpose(w2, (0, 2, 3, 1)).reshape(Co, 9 * Cm).astype(mxu_dtype)
    b1c = b1.reshape(Cm, 1)
    b2c = b2.reshape(Co, 1)

    n2 = 2 if N % 2 == 0 else 1
    y, sc2, sh2 = _fused_block(x3, skip3, sc1, sh1, w_taps, b_taps, perm,
                               w1mu, w1ms, b1c, w2m, b2c,
                               g2.reshape(Co, 1), be2.reshape(Co, 1),
                               H2, W2, n2, cnt, eps)

    out = _final_affine(y, sc2, sh2, n1)
    return out.reshape(N, Co, H2, W2)


# speedup vs baseline: 1.3354x; 1.0043x over previous
import functools

import numpy as np

import jax
import jax.numpy as jnp
from jax.experimental import pallas as pl
from jax.experimental.pallas import tpu as pltpu


def _stats_kernel(x_ref, w_ref, b_ref, g_ref, be_ref, sc_ref, sh_ref,
                  acc_ref, *, nimg, G, Co, cnt, eps):
    g = pl.program_id(0)

    @pl.when(g == 0)
    def _():
        acc_ref[...] = jnp.zeros_like(acc_ref)

    w = w_ref[...]
    b = b_ref[...]
    acc = None
    for i in range(nimg):
        x = x_ref[i].astype(jnp.bfloat16)
        z = jnp.dot(w, x, preferred_element_type=jnp.float32)
        z = jnp.maximum(z + b, 0.0)
        st = jnp.concatenate(
            [jnp.sum(z, axis=1, keepdims=True),
             jnp.sum(z * z, axis=1, keepdims=True)], axis=1)
        acc = st if acc is None else acc + st
    acc_ref[...] += acc

    @pl.when(g == G - 1)
    def _():
        st = acc_ref[...]
        stc = (st[0 * Co:1 * Co] + st[1 * Co:2 * Co]
               + st[2 * Co:3 * Co] + st[3 * Co:4 * Co])
        mean = stc[:, 0:1] / cnt
        var = jnp.maximum(stc[:, 1:2] / cnt - mean * mean, 0.0)
        scale = g_ref[...] * jax.lax.rsqrt(var + eps)
        shift = be_ref[...] - mean * scale
        sc_ref[...] = jnp.concatenate([scale] * 4, axis=0)
        sh_ref[...] = jnp.concatenate([shift] * 4, axis=0)


def _convt_stats(x3, w_taps, b_taps, g1c, be1c, nimg, cnt, eps):
    N, Ci, HWc = x3.shape
    Ko = w_taps.shape[0]
    Co = Ko // 4
    G = N // nimg
    body = functools.partial(_stats_kernel, nimg=nimg, G=G, Co=Co,
                             cnt=cnt, eps=eps)
    return pl.pallas_call(
        body,
        grid=(G,),
        in_specs=[
            pl.BlockSpec((nimg, Ci, HWc), lambda g: (g, 0, 0)),
            pl.BlockSpec((Ko, Ci), lambda g: (0, 0)),
            pl.BlockSpec((Ko, 1), lambda g: (0, 0)),
            pl.BlockSpec((Co, 1), lambda g: (0, 0)),
            pl.BlockSpec((Co, 1), lambda g: (0, 0)),
        ],
        out_specs=(
            pl.BlockSpec((Ko, 1), lambda g: (0, 0)),
            pl.BlockSpec((Ko, 1), lambda g: (0, 0)),
        ),
        out_shape=(
            jax.ShapeDtypeStruct((Ko, 1), jnp.float32),
            jax.ShapeDtypeStruct((Ko, 1), jnp.float32),
        ),
        scratch_shapes=[pltpu.VMEM((Ko, 2), jnp.float32)],
        compiler_params=pltpu.CompilerParams(
            dimension_semantics=("arbitrary",)),
    )(x3, w_taps, b_taps, g1c, be1c)


def _main_kernel(x_ref, skip_ref, sc1_ref, sh1_ref,
                 wt_ref, bt_ref, perm_ref,
                 w1u_ref, w1s_ref, b1_ref, w2_ref, b2_ref,
                 g2_ref, be2_ref,
                 y_ref, sc2_ref, sh2_ref, acc_ref,
                 *, Co, H, W, nimg, G, cnt, eps):
    HW = H * W
    step = pl.program_id(0)
    offs = [(dy, dx) for dy in (-1, 0, 1) for dx in (-1, 0, 1)]

    @pl.when(step == 0)
    def _():
        acc_ref[...] = jnp.zeros_like(acc_ref)

    col = jax.lax.broadcasted_iota(jnp.int32, (1, HW), 1)
    xi = col % W
    yi = col // W
    masks = {}
    for dy, dx in offs:
        if (dy, dx) == (0, 0):
            continue
        m = ((xi + dx >= 0) & (xi + dx < W) &
             (yi + dy >= 0) & (yi + dy < H))
        masks[(dy, dx)] = m.astype(jnp.bfloat16)

    def patches(img):
        taps = []
        for dy, dx in offs:
            if (dy, dx) == (0, 0):
                taps.append(img)
                continue
            s = dy * W + dx
            shifted = pltpu.roll(img, shift=(-s) % HW, axis=1)
            taps.append(shifted * masks[(dy, dx)])
        return jnp.concatenate(taps, axis=0)

    sc1 = sc1_ref[...]
    sh1 = sh1_ref[...]
    acc = None
    for i in range(nimg):
        x = x_ref[i].astype(jnp.bfloat16)
        z = jnp.dot(wt_ref[...], x, preferred_element_type=jnp.float32)
        z = jnp.maximum(z + bt_ref[...], 0.0)
        zn = (z * sc1 + sh1).astype(jnp.bfloat16)
        u = jnp.dot(zn[0 * Co:1 * Co], perm_ref[0],
                    preferred_element_type=jnp.float32)
        for p in range(1, 4):
            u = u + jnp.dot(zn[p * Co:(p + 1) * Co], perm_ref[p],
                            preferred_element_type=jnp.float32)
        u = u.astype(jnp.bfloat16)

        mid = (jnp.dot(w1u_ref[...], patches(u),
                       preferred_element_type=jnp.float32)
               + jnp.dot(w1s_ref[...], patches(skip_ref[i].astype(jnp.bfloat16)),
                         preferred_element_type=jnp.float32))
        mid = jnp.maximum(mid + b1_ref[...], 0.0).astype(jnp.bfloat16)
        y = jnp.dot(w2_ref[...], patches(mid),
                    preferred_element_type=jnp.float32)
        y = jnp.maximum(y + b2_ref[...], 0.0)

        y_ref[i] = y.astype(jnp.bfloat16)
        st = jnp.concatenate(
            [jnp.sum(y, axis=1, keepdims=True),
             jnp.sum(y * y, axis=1, keepdims=True)], axis=1)
        acc = st if acc is None else acc + st
    acc_ref[...] += acc

    @pl.when(step == G - 1)
    def _():
        st = acc_ref[...]
        mean = st[:, 0:1] / cnt
        var = jnp.maximum(st[:, 1:2] / cnt - mean * mean, 0.0)
        scale = g2_ref[...] * jax.lax.rsqrt(var + eps)
        sc2_ref[...] = scale
        sh2_ref[...] = be2_ref[...] - mean * scale


def _fused_block(x3, skip3, sc1, sh1, w_taps, b_taps, perm,
                 w1mu, w1ms, b1c, w2m, b2c, g2c, be2c,
                 H, W, nimg, cnt, eps):
    N, Ci, HWc = x3.shape
    Cs = skip3.shape[1]
    Ko = w_taps.shape[0]
    Co = Ko // 4
    Cm = w1mu.shape[0]
    HW = H * W
    G = N // nimg

    est = (2 * nimg * (Ci * HWc + Cs * HW + Co * HW) * 4
           + (Ko * Ci + Cm * 9 * (Co + Cs) + Co * 9 * Cm + 4 * HWc * HW) * 2
           + (9 * (Co + Cs) + 9 * Cm) * HW * 2
           + (Ko * HWc + Co * HW + Cm * HW) * 4)
    vmem_limit = int(min(max(3 * est, 32 * 2 ** 20), 100 * 2 ** 20))

    body = functools.partial(_main_kernel, Co=Co, H=H, W=W, nimg=nimg, G=G,
                             cnt=cnt, eps=eps)
    return pl.pallas_call(
        body,
        grid=(G,),
        in_specs=[
            pl.BlockSpec((nimg, Ci, HWc), lambda g: (g, 0, 0)),
            pl.BlockSpec((nimg, Cs, HW), lambda g: (g, 0, 0)),
            pl.BlockSpec((Ko, 1), lambda g: (0, 0)),
            pl.BlockSpec((Ko, 1), lambda g: (0, 0)),
            pl.BlockSpec((Ko, Ci), lambda g: (0, 0)),
            pl.BlockSpec((Ko, 1), lambda g: (0, 0)),
            pl.BlockSpec((4, HWc, HW), lambda g: (0, 0, 0)),
            pl.BlockSpec((Cm, 9 * Co), lambda g: (0, 0)),
            pl.BlockSpec((Cm, 9 * Cs), lambda g: (0, 0)),
            pl.BlockSpec((Cm, 1), lambda g: (0, 0)),
            pl.BlockSpec((Co, 9 * Cm), lambda g: (0, 0)),
            pl.BlockSpec((Co, 1), lambda g: (0, 0)),
            pl.BlockSpec((Co, 1), lambda g: (0, 0)),
            pl.BlockSpec((Co, 1), lambda g: (0, 0)),
        ],
        out_specs=(
            pl.BlockSpec((nimg, Co, HW), lambda g: (g, 0, 0)),
            pl.BlockSpec((Co, 1), lambda g: (0, 0)),
            pl.BlockSpec((Co, 1), lambda g: (0, 0)),
        ),
        out_shape=(
            jax.ShapeDtypeStruct((N, Co, HW), jnp.bfloat16),
            jax.ShapeDtypeStruct((Co, 1), jnp.float32),
            jax.ShapeDtypeStruct((Co, 1), jnp.float32),
        ),
        scratch_shapes=[pltpu.VMEM((Co, 2), jnp.float32)],
        compiler_params=pltpu.CompilerParams(
            dimension_semantics=("arbitrary",),
            vmem_limit_bytes=vmem_limit),
    )(x3, skip3, sc1, sh1, w_taps, b_taps, perm,
      w1mu, w1ms, b1c, w2m, b2c, g2c, be2c)


def _affine_kernel(y_ref, sc_ref, sh_ref, o_ref):
    o_ref[...] = (y_ref[...].astype(jnp.float32) * sc_ref[...][None]
                  + sh_ref[...][None])


def _final_affine(y, sc2, sh2, nimg):
    N, Co, HW = y.shape
    G = N // nimg
    return pl.pallas_call(
        _affine_kernel,
        grid=(G,),
        in_specs=[
            pl.BlockSpec((nimg, Co, HW), lambda g: (g, 0, 0)),
            pl.BlockSpec((Co, 1), lambda g: (0, 0)),
            pl.BlockSpec((Co, 1), lambda g: (0, 0)),
        ],
        out_specs=pl.BlockSpec((nimg, Co, HW), lambda g: (g, 0, 0)),
        out_shape=jax.ShapeDtypeStruct((N, Co, HW), jnp.float32),
        compiler_params=pltpu.CompilerParams(
            dimension_semantics=("arbitrary",)),
    )(y, sc2, sh2)


@functools.lru_cache(maxsize=4)
def _perm_mats(Hc, Wc):
    H2, W2 = 2 * Hc, 2 * Wc
    HWc = Hc * Wc
    P = np.zeros((4, HWc, H2 * W2), np.float32)
    hw = np.arange(HWc)
    h, w = hw // Wc, hw % Wc
    for dy in (0, 1):
        for dx in (0, 1):
            P[dy * 2 + dx, hw, (2 * h + dy) * W2 + 2 * w + dx] = 1.0
    return P


def kernel(x, skip, wt, bt, w1, b1, w2, b2, g1, be1, g2, be2):
    eps = 1e-5
    mxu_dtype = jnp.bfloat16
    x = x.astype(jnp.float32)
    skip = skip.astype(jnp.float32)
    N, Ci, Hc, Wc = x.shape
    _, Cs, H2, W2 = skip.shape
    Co = wt.shape[1]
    Cm = w1.shape[0]
    HWc, HW2 = Hc * Wc, H2 * W2

    x3 = x.reshape(N, Ci, HWc)
    skip3 = skip.reshape(N, Cs, HW2)

    w_taps = (jnp.transpose(wt, (2, 3, 1, 0))
              .reshape(4 * Co, Ci).astype(mxu_dtype))
    b_taps = jnp.tile(bt, 4).reshape(4 * Co, 1)

    cnt = N * HW2
    n1 = 8 if N % 8 == 0 else 1
    sc1, sh1 = _convt_stats(x3, w_taps, b_taps,
                            g1.reshape(Co, 1), be1.reshape(Co, 1),
                            n1, cnt, eps)

    perm = jnp.asarray(_perm_mats(Hc, Wc), dtype=mxu_dtype)
    w1mu = (jnp.transpose(w1[:, :Co], (0, 2, 3, 1))
            .reshape(Cm, 9 * Co).astype(mxu_dtype))
    w1ms = (jnp.transpose(w1[:, Co:], (0, 2, 3, 1))
            .reshape(Cm, 9 * Cs).astype(mxu_dtype))
    w2m = jnp.transpose(w2, (0, 2, 3, 1)).reshape(Co, 9 * Cm).astype(mxu_dtype)
    b1c = b1.reshape(Cm, 1)
    b2c = b2.reshape(Co, 1)

    n2 = 2 if N % 2 == 0 else 1
    y, sc2, sh2 = _fused_block(x3, skip3, sc1, sh1, w_taps, b_taps, perm,
                               w1mu, w1ms, b1c, w2m, b2c,
                               g2.reshape(Co, 1), be2.reshape(Co, 1),
                               H2, W2, n2, cnt, eps)

    n3 = 16 if N % 16 == 0 else n1
    out = _final_affine(y, sc2, sh2, n3)
    return out.reshape(N, Co, H2, W2)
